# baseline (speedup 1.0000x reference)
import functools

import jax
import jax.numpy as jnp
from jax import lax
from jax.experimental import pallas as pl
from jax.experimental.pallas import tpu as pltpu
from jax.experimental.pallas import tpu_sc as plsc

_SEGMENT_LENGTH = 512
_NUM_SEGMENTS = 32


def _segment_starts(n: int) -> list[int]:
    l, s = _SEGMENT_LENGTH, _NUM_SEGMENTS
    starts = []
    for i in range(s):
        st = i * (n - l) // max(1, s - 1)
        if st + l > n:
            st = n - l
        starts.append(st)
    return starts


@jax.jit
def kernel(points):
    b_dim, n, c = points.shape
    l, s = _SEGMENT_LENGTH, _NUM_SEGMENTS
    starts = _segment_starts(n)
    seg_w = l * c

    info = plsc.get_sparse_core_info()
    nc, ns = info.num_cores, info.num_subcores
    nw = nc * ns
    b_per_w = b_dim // nw

    mesh = plsc.VectorSubcoreMesh(core_axis_name="c", subcore_axis_name="s")

    @functools.partial(
        pl.kernel,
        mesh=mesh,
        out_type=jax.ShapeDtypeStruct((b_dim, s, l, c), points.dtype),
        scratch_types=[
            pltpu.SemaphoreType.DMA,
        ],
        compiler_params=pltpu.CompilerParams(use_tc_tiling_on_sc=False),
    )
    def seg_sampler(points_hbm, out_hbm, sem):
        wid = lax.axis_index("s") * nc + lax.axis_index("c")
        base = wid * b_per_w
        copies = []
        for db in range(b_per_w):
            bi = base + db
            for si, st in enumerate(starts):
                cp = pltpu.make_async_copy(
                    points_hbm.at[bi, pl.ds(st, l), :],
                    out_hbm.at[bi, si],
                    sem,
                )
                cp.start()
                copies.append(cp)
        for cp in copies:
            cp.wait()

    return seg_sampler(points)


# speedup vs baseline: 53.5911x; 53.5911x over previous
import functools

import jax
import jax.numpy as jnp
from jax import lax
from jax.experimental import pallas as pl
from jax.experimental.pallas import tpu as pltpu
from jax.experimental.pallas import tpu_sc as plsc

_SEGMENT_LENGTH = 512
_NUM_SEGMENTS = 32


def _segment_starts(n: int) -> list[int]:
    l, s = _SEGMENT_LENGTH, _NUM_SEGMENTS
    starts = []
    for i in range(s):
        st = i * (n - l) // max(1, s - 1)
        if st + l > n:
            st = n - l
        starts.append(st)
    return starts


@jax.jit
def kernel(points):
    b_dim, n, c = points.shape
    l, s = _SEGMENT_LENGTH, _NUM_SEGMENTS
    starts = _segment_starts(n)
    seg_w = l * c

    spans = []
    for st in starts:
        e0 = st * c
        a0 = (e0 // 8) * 8
        off = e0 - a0
        w = seg_w + (8 if off else 0)
        spans.append((a0, off, w))

    info = plsc.get_sparse_core_info()
    nc, ns = info.num_cores, info.num_subcores
    nw = nc * ns
    b_per_w = b_dim // nw

    mesh = plsc.VectorSubcoreMesh(core_axis_name="c", subcore_axis_name="s")

    @functools.partial(
        pl.kernel,
        mesh=mesh,
        out_type=jax.ShapeDtypeStruct((b_dim, s, seg_w), points.dtype),
        scratch_types=[
            pltpu.VMEM((s, seg_w + 8), points.dtype),
            pltpu.VMEM((s, seg_w), points.dtype),
            pltpu.SemaphoreType.DMA,
            pltpu.SemaphoreType.DMA,
        ],
        compiler_params=pltpu.CompilerParams(use_tc_tiling_on_sc=False),
    )
    def seg_sampler(points_hbm, out_hbm, buf, outbuf, sem, wsem):
        wid = lax.axis_index("s") * nc + lax.axis_index("c")
        base = wid * b_per_w
        for db in range(b_per_w):
            bi = base + db
            copies = []
            for si, (a0, off, w) in enumerate(spans):
                if off == 0:
                    cp = pltpu.make_async_copy(
                        points_hbm.at[bi, pl.ds(a0, seg_w)],
                        outbuf.at[si],
                        sem,
                    )
                else:
                    cp = pltpu.make_async_copy(
                        points_hbm.at[bi, pl.ds(a0, w)],
                        buf.at[si, pl.ds(0, w)],
                        sem,
                    )
                cp.start()
                copies.append(cp)
            for cp in copies:
                cp.wait()
            for si, (a0, off, w) in enumerate(spans):
                if off == 0:
                    continue

                def shift_body(k, _, si=si, off=off):
                    outbuf[si, pl.ds(k * 16, 16)] = buf[si, pl.ds(off + k * 16, 16)]
                    return _

                lax.fori_loop(0, seg_w // 16, shift_body, None, unroll=8)
            wb = pltpu.make_async_copy(outbuf, out_hbm.at[bi], wsem)
            wb.start()
            wb.wait()

    out = seg_sampler(points.reshape(b_dim, n * c))
    return out.reshape(b_dim, s, l, c)


# speedup vs baseline: 247.9769x; 4.6272x over previous
import functools

import jax
import jax.numpy as jnp
from jax import lax
from jax.experimental import pallas as pl
from jax.experimental.pallas import tpu as pltpu
from jax.experimental.pallas import tpu_sc as plsc

_SEGMENT_LENGTH = 512
_NUM_SEGMENTS = 32
_LANE_TILE = 128


def _segment_starts(n: int) -> list[int]:
    l, s = _SEGMENT_LENGTH, _NUM_SEGMENTS
    starts = []
    for i in range(s):
        st = i * (n - l) // max(1, s - 1)
        if st + l > n:
            st = n - l
        starts.append(st)
    return starts


@jax.jit
def kernel(points):
    b_dim, n, c = points.shape
    l, s = _SEGMENT_LENGTH, _NUM_SEGMENTS
    starts = _segment_starts(n)
    buf_w = l + _LANE_TILE

    n_al = (n // _LANE_TILE) * _LANE_TILE
    tail_w = n - n_al

    spans = []
    for st in starts:
        a0 = (st // _LANE_TILE) * _LANE_TILE
        off = st - a0
        end = a0 + (buf_w if off else l)
        tail = max(0, min(end, st + l) - n_al)
        w = min(end, n_al) - a0
        spans.append((a0, off, w, tail))

    info = plsc.get_sparse_core_info()
    nc, ns = info.num_cores, info.num_subcores
    nw = nc * ns
    b_per_w = b_dim // nw

    mesh = plsc.VectorSubcoreMesh(core_axis_name="c", subcore_axis_name="s")

    @functools.partial(
        pl.kernel,
        mesh=mesh,
        out_type=jax.ShapeDtypeStruct((b_dim, s, c, l), points.dtype),
        scratch_types=[
            pltpu.VMEM((s, c, buf_w), points.dtype),
            pltpu.VMEM((s, c, l), points.dtype),
            pltpu.VMEM((c, tail_w), points.dtype),
            pltpu.SemaphoreType.DMA,
            pltpu.SemaphoreType.DMA,
        ],
        compiler_params=pltpu.CompilerParams(
            use_tc_tiling_on_sc=True, needs_layout_passes=False
        ),
    )
    def seg_sampler(points_hbm, tail_hbm, out_hbm, buf, outbuf, tbuf, sem, wsem):
        wid = lax.axis_index("s") * nc + lax.axis_index("c")
        base = wid * b_per_w
        for db in range(b_per_w):
            bi = base + db
            copies = []
            for si, (a0, off, w, tail) in enumerate(spans):
                if off == 0:
                    cp = pltpu.make_async_copy(
                        points_hbm.at[bi, :, pl.ds(a0, l)],
                        outbuf.at[si],
                        sem,
                    )
                else:
                    cp = pltpu.make_async_copy(
                        points_hbm.at[bi, :, pl.ds(a0, w)],
                        buf.at[si, :, pl.ds(0, w)],
                        sem,
                    )
                cp.start()
                copies.append(cp)
            tp = pltpu.make_async_copy(tail_hbm.at[bi], tbuf, sem)
            tp.start()
            copies.append(tp)
            for cp in copies:
                cp.wait()
            lane = lax.iota(jnp.int32, 16)
            for si, (a0, off, w, tail) in enumerate(spans):
                if off == 0:
                    continue
                main = l - tail
                idx_base = lane + off

                si_v = jnp.full((16,), si, jnp.int32)
                zero_v = jnp.zeros((16,), jnp.int32)
                one_v = jnp.ones((16,), jnp.int32)

                def shift_body(k, _, si_v=si_v, idx_base=idx_base):
                    idx = idx_base + k * 16
                    v0 = plsc.load_gather(buf, [si_v, zero_v, idx])
                    v1 = plsc.load_gather(buf, [si_v, one_v, idx])
                    outbuf[si, 0, pl.ds(k * 16, 16)] = v0
                    outbuf[si, 1, pl.ds(k * 16, 16)] = v1
                    return _

                lax.fori_loop(0, main // 16, shift_body, None, unroll=4)
                for j in range(main, l, 16):
                    outbuf[si, 0, pl.ds(j, 16)] = tbuf[0, pl.ds(j - main, 16)]
                    outbuf[si, 1, pl.ds(j, 16)] = tbuf[1, pl.ds(j - main, 16)]
            wb = pltpu.make_async_copy(outbuf, out_hbm.at[bi], wsem)
            wb.start()
            wb.wait()

    tail_in = points[:, n_al:, :].transpose(0, 2, 1)
    out = seg_sampler(points.transpose(0, 2, 1), tail_in)
    return out.transpose(0, 1, 3, 2)


# speedup vs baseline: 252.7110x; 1.0191x over previous
import functools

import jax
import jax.numpy as jnp
from jax import lax
from jax.experimental import pallas as pl
from jax.experimental.pallas import tpu as pltpu
from jax.experimental.pallas import tpu_sc as plsc

_SEGMENT_LENGTH = 512
_NUM_SEGMENTS = 32
_LANE_TILE = 128


def _segment_starts(n: int) -> list[int]:
    l, s = _SEGMENT_LENGTH, _NUM_SEGMENTS
    starts = []
    for i in range(s):
        st = i * (n - l) // max(1, s - 1)
        if st + l > n:
            st = n - l
        starts.append(st)
    return starts


@jax.jit
def kernel(points):
    b_dim, n, c = points.shape
    l, s = _SEGMENT_LENGTH, _NUM_SEGMENTS
    starts = _segment_starts(n)
    buf_w = l + _LANE_TILE

    n_al = (n // _LANE_TILE) * _LANE_TILE
    tail_w = n - n_al

    spans = []
    for st in starts:
        a0 = (st // _LANE_TILE) * _LANE_TILE
        off = st - a0
        end = a0 + (buf_w if off else l)
        tail = max(0, min(end, st + l) - n_al)
        w = min(end, n_al) - a0
        spans.append((a0, off, w, tail))

    info = plsc.get_sparse_core_info()
    nc, ns = info.num_cores, info.num_subcores
    nw = nc * ns
    b_per_w = b_dim // nw

    mesh = plsc.VectorSubcoreMesh(core_axis_name="c", subcore_axis_name="s")

    @functools.partial(
        pl.kernel,
        mesh=mesh,
        out_type=jax.ShapeDtypeStruct((b_dim, s, c, l), points.dtype),
        scratch_types=[
            pltpu.VMEM((s, c, buf_w), points.dtype),
            pltpu.VMEM((s, c, buf_w), points.dtype),
            pltpu.VMEM((s, c, l), points.dtype),
            pltpu.VMEM((b_per_w, c, tail_w), points.dtype),
            pltpu.SemaphoreType.DMA,
            pltpu.SemaphoreType.DMA,
            pltpu.SemaphoreType.DMA,
        ],
        compiler_params=pltpu.CompilerParams(
            use_tc_tiling_on_sc=True, needs_layout_passes=False
        ),
    )
    def seg_sampler(
        points_hbm, tail_hbm, out_hbm, buf0, buf1, outbuf, tbuf, sem0, sem1, wsem
    ):
        wid = lax.axis_index("s") * nc + lax.axis_index("c")
        base = wid * b_per_w
        bufs = (buf0, buf1)
        sems = (sem0, sem1)

        def start_gathers(bi, buf, sem):
            cps = []
            for si, (a0, off, w, tail) in enumerate(spans):
                cp = pltpu.make_async_copy(
                    points_hbm.at[bi, :, pl.ds(a0, w)],
                    buf.at[si, :, pl.ds(0, w)],
                    sem,
                )
                cp.start()
                cps.append(cp)
            return cps

        def realign(buf, db):
            lane = lax.iota(jnp.int32, 16)
            zero_v = jnp.zeros((16,), jnp.int32)
            one_v = jnp.ones((16,), jnp.int32)
            for si, (a0, off, w, tail) in enumerate(spans):
                main = l - tail
                idx_base = lane + off
                si_v = jnp.full((16,), si, jnp.int32)

                def shift_body(k, _, si=si, si_v=si_v, idx_base=idx_base, buf=buf):
                    idx = idx_base + k * 16
                    v0 = plsc.load_gather(buf, [si_v, zero_v, idx])
                    v1 = plsc.load_gather(buf, [si_v, one_v, idx])
                    outbuf[si, 0, pl.ds(k * 16, 16)] = v0
                    outbuf[si, 1, pl.ds(k * 16, 16)] = v1
                    return _

                lax.fori_loop(0, main // 16, shift_body, None, unroll=4)
                for j in range(main, l, 16):
                    outbuf[si, 0, pl.ds(j, 16)] = tbuf[db, 0, pl.ds(j - main, 16)]
                    outbuf[si, 1, pl.ds(j, 16)] = tbuf[db, 1, pl.ds(j - main, 16)]

        tp = pltpu.make_async_copy(
            tail_hbm.at[pl.ds(base, b_per_w)], tbuf, wsem
        )
        tp.start()
        gathers = [
            start_gathers(base + db, bufs[db % 2], sems[db % 2])
            for db in range(b_per_w)
        ]
        tp.wait()
        prev_wb = None
        for db in range(b_per_w):
            for cp in gathers[db]:
                cp.wait()
            if prev_wb is not None:
                prev_wb.wait()
            realign(bufs[db % 2], db)
            wb = pltpu.make_async_copy(outbuf, out_hbm.at[base + db], wsem)
            wb.start()
            prev_wb = wb
        prev_wb.wait()

    tail_in = points[:, n_al:, :].transpose(0, 2, 1)
    out = seg_sampler(points.transpose(0, 2, 1), tail_in)
    return out.transpose(0, 1, 3, 2)
